# baseline (speedup 1.0000x reference)
import functools

import jax
import jax.numpy as jnp
from jax import lax
from jax.experimental import pallas as pl
from jax.experimental.pallas import tpu as pltpu
from jax.experimental.pallas import tpu_sc as plsc

NC = 2
NS = 16
NW = NC * NS

T = 256
NBUF = 4
L = 16


@functools.partial(jax.jit, static_argnames=("n_rows", "d"))
def _sc_gather(lines, coloffs, table_flat, *, n_rows, d):
    rows_per_w = n_rows // NW
    streams_per_w = rows_per_w // T
    ngroups = streams_per_w // NBUF
    n_lines = table_flat.shape[0] // 64

    mesh = plsc.VectorSubcoreMesh(core_axis_name="c", subcore_axis_name="s")

    @functools.partial(
        pl.kernel,
        out_type=jax.ShapeDtypeStruct((n_rows * d,), jnp.float32),
        mesh=mesh,
        compiler_params=pltpu.CompilerParams(
            use_tc_tiling_on_sc=False, needs_layout_passes=False),
        scratch_types=[
            pltpu.VMEM((NBUF, 2, T), jnp.int32),
            pltpu.VMEM((NBUF, T, 64), jnp.float32),
            pltpu.VMEM((2, T * d), jnp.float32),
            pltpu.SemaphoreType.DMA((NBUF, 2)),
            pltpu.SemaphoreType.DMA((NBUF,)),
            pltpu.SemaphoreType.DMA((2,)),
        ],
    )
    def gather_kernel(lin_hbm, col_hbm, table_hbm, out_hbm, idx_v, gat_v,
                      sel_v, idx_sem, gat_sem, out_sem):
        wid = lax.axis_index("s") * NC + lax.axis_index("c")
        base = wid * rows_per_w

        def lin_copy(s, b):
            return pltpu.make_async_copy(
                lin_hbm.at[pl.ds(base + s * T, T)], idx_v.at[b, 0],
                idx_sem.at[b, 0])

        def col_copy(s, b):
            return pltpu.make_async_copy(
                col_hbm.at[pl.ds(base + s * T, T)], idx_v.at[b, 1],
                idx_sem.at[b, 1])

        def gat_copy(b):
            return pltpu.make_async_copy(
                table_hbm.at[idx_v.at[b, 0]], gat_v.at[b], gat_sem.at[b])

        def out_copy(s, p):
            return pltpu.make_async_copy(
                sel_v.at[p], out_hbm.at[pl.ds((base + s * T) * d, T * d)],
                out_sem.at[p])

        def select(b, p):
            gat = gat_v.at[b]
            sel = sel_v.at[p]
            iota = lax.iota(jnp.int32, L)

            def group(k, carry):
                row_ids = iota + k * L
                coloff = idx_v[b, 1, pl.ds(k * L, L)]
                dst0 = row_ids * d
                for j in range(d):
                    x = plsc.load_gather(gat, [row_ids, coloff + j])
                    plsc.store_scatter(sel, [dst0 + j], x)
                return carry

            lax.fori_loop(0, T // L, group, 0)

        for b in range(NBUF):
            lin_copy(b, b).start()
            col_copy(b, b).start()

        def body(g, carry):
            s0 = g * NBUF
            for b in range(NBUF):
                lin_copy(s0 + b, b).wait()
                gat_copy(b).start()
            for b in range(NBUF):
                p = b % 2
                gat_copy(b).wait()
                col_copy(s0 + b, b).wait()
                if b >= 2:
                    out_copy(s0 + b, p).wait()
                else:
                    pl.when(g > 0)(lambda b=b, p=p, s0=s0:
                                   out_copy(s0 + b, p).wait())
                select(b, p)
                out_copy(s0 + b, p).start()
                pl.when(g < ngroups - 1)(
                    lambda b=b, s0=s0: lin_copy(s0 + NBUF + b, b).start())
                pl.when(g < ngroups - 1)(
                    lambda b=b, s0=s0: col_copy(s0 + NBUF + b, b).start())
            return carry

        lax.fori_loop(0, ngroups, body, 0)

        for b in range(NBUF - 2, NBUF):
            out_copy((ngroups - 1) * NBUF + b, b % 2).wait()

    return gather_kernel(lines, coloffs, table_flat)


def kernel(inputs, tables):
    f, v, d = tables.shape
    b = inputs.shape[0]
    n_rows = b * f

    offsets = (jnp.arange(f, dtype=jnp.int32) * v)[None, :]
    flat = (inputs + offsets).reshape(n_rows)
    lines = flat >> 1
    coloffs = (flat & 1) * d
    table_flat = tables.reshape(f * v * d // 64, 64)

    out = _sc_gather(lines, coloffs, table_flat, n_rows=n_rows, d=d)
    return out.reshape(b, f, d)


# speedup vs baseline: 1.0069x; 1.0069x over previous
import functools

import jax
import jax.numpy as jnp
from jax import lax
from jax.experimental import pallas as pl
from jax.experimental.pallas import tpu as pltpu
from jax.experimental.pallas import tpu_sc as plsc

NC = 2
NS = 16
NW = NC * NS

T = 256
NBUF = 4
L = 16


@functools.partial(jax.jit, static_argnames=("n_rows", "d"))
def _sc_gather(lines, coloffs, table_flat, *, n_rows, d):
    rows_per_w = n_rows // NW
    streams_per_w = rows_per_w // T
    ngroups = streams_per_w // NBUF
    n_lines = table_flat.shape[0] // 64

    mesh = plsc.VectorSubcoreMesh(core_axis_name="c", subcore_axis_name="s")

    @functools.partial(
        pl.kernel,
        out_type=jax.ShapeDtypeStruct((n_rows * d,), jnp.float32),
        mesh=mesh,
        compiler_params=pltpu.CompilerParams(
            use_tc_tiling_on_sc=False, needs_layout_passes=False),
        scratch_types=[
            pltpu.VMEM((NBUF, 2, T), jnp.int32),
            pltpu.VMEM((NBUF, T, 64), jnp.float32),
            pltpu.VMEM((2, T * d), jnp.float32),
            pltpu.SemaphoreType.DMA((NBUF, 2)),
            pltpu.SemaphoreType.DMA((NBUF,)),
            pltpu.SemaphoreType.DMA((2,)),
        ],
    )
    def gather_kernel(lin_hbm, col_hbm, table_hbm, out_hbm, idx_v, gat_v,
                      sel_v, idx_sem, gat_sem, out_sem):
        wid = lax.axis_index("s") * NC + lax.axis_index("c")
        base = wid * rows_per_w

        def lin_copy(s, b):
            return pltpu.make_async_copy(
                lin_hbm.at[pl.ds(base + s * T, T)], idx_v.at[b, 0],
                idx_sem.at[b, 0])

        def col_copy(s, b):
            return pltpu.make_async_copy(
                col_hbm.at[pl.ds(base + s * T, T)], idx_v.at[b, 1],
                idx_sem.at[b, 1])

        def gat_copy(b):
            return pltpu.make_async_copy(
                table_hbm.at[idx_v.at[b, 0]], gat_v.at[b], gat_sem.at[b])

        def out_copy(s, p):
            return pltpu.make_async_copy(
                sel_v.at[p], out_hbm.at[pl.ds((base + s * T) * d, T * d)],
                out_sem.at[p])

        def select(b, p):
            gat = gat_v.at[b]
            sel = sel_v.at[p]
            iota = lax.iota(jnp.int32, L)

            def group(k, carry):
                row_ids = iota + k * L
                coloff = idx_v[b, 1, pl.ds(k * L, L)]
                dst0 = row_ids * d
                for j in range(d):
                    x = plsc.load_gather(gat, [row_ids, coloff + j])
                    plsc.store_scatter(sel, [dst0 + j], x)
                return carry

            lax.fori_loop(0, T // L, group, 0)

        for b in range(NBUF):
            lin_copy(b, b).start()
            col_copy(b, b).start()
        for b in range(NBUF):
            lin_copy(b, b).wait()
            gat_copy(b).start()

        def body(g, carry):
            s0 = g * NBUF
            for b in range(NBUF):
                p = b % 2
                gat_copy(b).wait()
                pl.when(g < ngroups - 1)(
                    lambda b=b, s0=s0: lin_copy(s0 + NBUF + b, b).start())
                col_copy(s0 + b, b).wait()
                if b >= 2:
                    out_copy(s0 + b, p).wait()
                else:
                    pl.when(g > 0)(lambda b=b, p=p, s0=s0:
                                   out_copy(s0 + b, p).wait())
                select(b, p)
                out_copy(s0 + b, p).start()
                def refill(b=b, s0=s0):
                    lin_copy(s0 + NBUF + b, b).wait()
                    col_copy(s0 + NBUF + b, b).start()
                    gat_copy(b).start()
                pl.when(g < ngroups - 1)(refill)
            return carry

        lax.fori_loop(0, ngroups, body, 0)

        for b in range(NBUF - 2, NBUF):
            out_copy((ngroups - 1) * NBUF + b, b % 2).wait()

    return gather_kernel(lines, coloffs, table_flat)


def kernel(inputs, tables):
    f, v, d = tables.shape
    b = inputs.shape[0]
    n_rows = b * f

    offsets = (jnp.arange(f, dtype=jnp.int32) * v)[None, :]
    flat = (inputs + offsets).reshape(n_rows)
    lines = flat >> 1
    coloffs = (flat & 1) * d
    table_flat = tables.reshape(f * v * d // 64, 64)

    out = _sc_gather(lines, coloffs, table_flat, n_rows=n_rows, d=d)
    return out.reshape(b, f, d)
